# baseline (speedup 1.0000x reference)
import functools

import jax
import jax.numpy as jnp
from jax import lax
from jax.experimental import pallas as pl
from jax.experimental.pallas import tpu as pltpu
from jax.experimental.pallas import tpu_sc as plsc

STATE = 128
START = 126
END = 127
B = 256
L = 512
LANES = 16
NC, NS = 2, 16
NW = NC * NS
SPW = B // NW
NGATHER = 4
GLEN = L // NGATHER
CHUNKS_PER_G = GLEN // LANES
NCHUNK = L // LANES
UNROLL = 4

_GATHER_DNUMS = lax.GatherDimensionNumbers(
    offset_dims=(), collapsed_slice_dims=(0,), start_index_map=(0,))


def _permute(v, idx):
    return lax.gather(v, idx[:, None], _GATHER_DNUMS, slice_sizes=(1,),
                      mode=lax.GatherScatterMode.PROMISE_IN_BOUNDS)


def _lanesum_splat(v):
    for sh in (8, 4, 2, 1):
        v = v + _permute(v, jnp.bitwise_xor(lax.iota(jnp.int32, LANES), sh))
    return v


@functools.partial(
    pl.kernel,
    out_type=jax.ShapeDtypeStruct((B,), jnp.float32),
    mesh=plsc.VectorSubcoreMesh(core_axis_name="c", subcore_axis_name="s"),
    compiler_params=pltpu.CompilerParams(needs_layout_passes=False),
    scratch_types=[
        pltpu.VMEM((STATE, STATE), jnp.float32),
        pltpu.VMEM((2 * SPW, L), jnp.int32),
        pltpu.VMEM((2, NGATHER, GLEN), jnp.int32),
        pltpu.VMEM((2, L), jnp.float32),
        pltpu.VMEM((LANES,), jnp.int32),
        pltpu.VMEM((LANES,), jnp.float32),
        pltpu.SemaphoreType.DMA,
        pltpu.SemaphoreType.DMA,
    ],
)
def _score(feats_hbm, w_hbm, tags_hbm, len_hbm, out_hbm,
           w_v, tags_v, idx_v, vals_v, len_v, out_v, sem0, sem1):
    wid = lax.axis_index("s") * NC + lax.axis_index("c")
    base_b = wid * SPW
    iota = lax.iota(jnp.int32, LANES)
    iota_s = iota * STATE
    sems = (sem0, sem1)
    zero16 = jnp.zeros((LANES,), jnp.int32)

    pltpu.sync_copy(w_hbm, w_v)
    pltpu.sync_copy(len_hbm.at[pl.ds(base_b, SPW)], len_v.at[pl.ds(0, SPW)])
    pltpu.sync_copy(tags_hbm.at[pl.ds(base_b, SPW)], tags_v.at[pl.ds(SPW, SPW)])
    lenvec = len_v[...].astype(jnp.float32)

    for p in range(2):
        for g in range(NGATHER):
            def clear(c, carry, p=p, g=g):
                idx_v[p, g, pl.ds(c * LANES, LANES)] = zero16
                return carry
            lax.fori_loop(0, CHUNKS_PER_G, clear, 0, unroll=8)

    def lens_of(i):
        l = _lanesum_splat(jnp.where(iota == i, lenvec, 0.0)).astype(jnp.int32)
        l_s = lax.squeeze(lax.slice(l, (0,), (1,)), (0,))
        return l, l_s, l_s // LANES + 1

    def build_and_fire(i, p, l_s, nchunks):
        fbase = (base_b + i) * (L * STATE)
        ng = l_s // GLEN
        for g in range(NGATHER):
            gbase = fbase + g * (GLEN * STATE)

            def build(c, carry, g=g, gbase=gbase):
                off = c * LANES
                tt = tags_v[i + SPW, pl.ds(g * GLEN + off, LANES)]
                idx_v[p, g, pl.ds(off, LANES)] = (gbase + off * STATE) + (iota_s + tt)
                return carry

            nb = jnp.clip(nchunks - g * CHUNKS_PER_G, 0, CHUNKS_PER_G)
            lax.fori_loop(0, nb, build, 0)

            @pl.when(g <= ng)
            def _fire(g=g):
                pltpu.async_copy(feats_hbm.at[idx_v.at[p, g]],
                                 vals_v.at[p, pl.ds(g * GLEN, GLEN)], sems[p])

    def drain(i, p, l_s):
        ng = l_s // GLEN
        for g in range(NGATHER):
            @pl.when(g <= ng)
            def _wait(g=g):
                pltpu.make_async_copy(feats_hbm.at[idx_v.at[p, g]],
                                      vals_v.at[p, pl.ds(g * GLEN, GLEN)],
                                      sems[p]).wait()

    linfo = {0: lens_of(0)}
    build_and_fire(0, 0, linfo[0][1], linfo[0][2])
    outvec = jnp.zeros((LANES,), jnp.float32)
    for i in range(SPW):
        p = i % 2
        if i + 1 < SPW:
            linfo[i + 1] = lens_of(i + 1)
            build_and_fire(i + 1, 1 - p, linfo[i + 1][1], linfo[i + 1][2])

        l, l_s, nchunks = linfo.pop(i)
        drain(i, p, l_s)

        rowidx = jnp.full((LANES,), i + SPW, jnp.int32)

        def block(blk, acc, p=p, rowidx=rowidx, l=l, i=i):
            for u in range(UNROLL):
                c = blk * UNROLL + u
                pos = c * LANES + iota
                tt = tags_v[i + SPW, pl.ds(c * LANES, LANES)]
                prev = plsc.load_gather(tags_v, [rowidx, jnp.maximum(pos - 1, 0)])
                col = jnp.where(pos == 0, START, prev)
                row = jnp.where(pos == l, END, tt)
                wv = plsc.load_gather(w_v, [row, col])
                v = vals_v[p, pl.ds(c * LANES, LANES)]
                acc = (acc + jnp.where(pos <= l, wv, 0.0)
                       + jnp.where(pos < l, v, 0.0))
            return acc

        nblocks = (nchunks + UNROLL - 1) // UNROLL
        acc = lax.fori_loop(0, nblocks, block, jnp.zeros((LANES,), jnp.float32))
        outvec = jnp.where(iota == i, _lanesum_splat(acc), outvec)

    out_v[...] = outvec
    pltpu.sync_copy(out_v.at[pl.ds(0, SPW)], out_hbm.at[pl.ds(base_b, SPW)])


def kernel(feats, weights, tags, length):
    return _score(feats.reshape(-1), weights, tags, length)


# speedup vs baseline: 2.0378x; 2.0378x over previous
import functools

import jax
import jax.numpy as jnp
from jax import lax
from jax.experimental import pallas as pl
from jax.experimental.pallas import tpu as pltpu
from jax.experimental.pallas import tpu_sc as plsc

STATE = 128
START = 126
END = 127
B = 256
L = 512
LANES = 16
NC, NS = 2, 16
NW = NC * NS
SPW = B // NW
NGATHER = 4
GLEN = L // NGATHER
CHUNKS_PER_G = GLEN // LANES
NCHUNK = L // LANES
UNROLL = 4

_GATHER_DNUMS = lax.GatherDimensionNumbers(
    offset_dims=(), collapsed_slice_dims=(0,), start_index_map=(0,))


def _permute(v, idx):
    return lax.gather(v, idx[:, None], _GATHER_DNUMS, slice_sizes=(1,),
                      mode=lax.GatherScatterMode.PROMISE_IN_BOUNDS)


def _lanesum_splat(v):
    for sh in (8, 4, 2, 1):
        v = v + _permute(v, jnp.bitwise_xor(lax.iota(jnp.int32, LANES), sh))
    return v


@functools.partial(
    pl.kernel,
    out_type=jax.ShapeDtypeStruct((B,), jnp.float32),
    mesh=plsc.VectorSubcoreMesh(core_axis_name="c", subcore_axis_name="s"),
    compiler_params=pltpu.CompilerParams(needs_layout_passes=False),
    scratch_types=[
        pltpu.VMEM((STATE, STATE), jnp.float32),
        pltpu.VMEM((2 * SPW, L), jnp.int32),
        pltpu.VMEM((2, NGATHER, GLEN), jnp.int32),
        pltpu.VMEM((2, L), jnp.float32),
        pltpu.VMEM((LANES,), jnp.int32),
        pltpu.VMEM((LANES,), jnp.float32),
        pltpu.SemaphoreType.DMA,
        pltpu.SemaphoreType.DMA,
    ],
)
def _score(feats_hbm, w_hbm, tags_hbm, len_hbm, out_hbm,
           w_v, tags_v, idx_v, vals_v, len_v, out_v, sem0, sem1):
    wid = lax.axis_index("s") * NC + lax.axis_index("c")
    base_b = wid * SPW
    iota = lax.iota(jnp.int32, LANES)
    iota_s = iota * STATE
    sems = (sem0, sem1)
    pltpu.sync_copy(w_hbm, w_v)
    pltpu.sync_copy(len_hbm.at[pl.ds(base_b, SPW)], len_v.at[pl.ds(0, SPW)])
    pltpu.sync_copy(tags_hbm.at[pl.ds(base_b, SPW)], tags_v.at[pl.ds(SPW, SPW)])
    lenvec = len_v[...].astype(jnp.float32)

    def build_and_fire(i, p):
        fbase = (base_b + i) * (L * STATE)
        for g in range(NGATHER):
            gbase = fbase + g * (GLEN * STATE)

            def build(c, carry, g=g, gbase=gbase):
                off = c * LANES
                tt = tags_v[i + SPW, pl.ds(g * GLEN + off, LANES)]
                idx_v[p, g, pl.ds(off, LANES)] = (gbase + off * STATE) + (iota_s + tt)
                return carry

            lax.fori_loop(0, CHUNKS_PER_G, build, 0, unroll=4)
        return [
            pltpu.async_copy(feats_hbm.at[idx_v.at[p, g]],
                             vals_v.at[p, pl.ds(g * GLEN, GLEN)], sems[p])
            for g in range(NGATHER)
        ]

    copies = {0: build_and_fire(0, 0)}
    outvec = jnp.zeros((LANES,), jnp.float32)
    for i in range(SPW):
        p = i % 2
        if i + 1 < SPW:
            copies[i + 1] = build_and_fire(i + 1, 1 - p)

        l = _lanesum_splat(jnp.where(iota == i, lenvec, 0.0)).astype(jnp.int32)

        for cp in copies.pop(i):
            cp.wait()

        rowidx = jnp.full((LANES,), i + SPW, jnp.int32)

        def fused(c, acc, p=p, rowidx=rowidx, l=l, i=i):
            pos = c * LANES + iota
            tt = tags_v[i + SPW, pl.ds(c * LANES, LANES)]
            prev = plsc.load_gather(tags_v, [rowidx, jnp.maximum(pos - 1, 0)])
            col = jnp.where(pos == 0, START, prev)
            row = jnp.where(pos == l, END, tt)
            wv = plsc.load_gather(w_v, [row, col])
            v = vals_v[p, pl.ds(c * LANES, LANES)]
            return (acc + jnp.where(pos <= l, wv, 0.0)
                    + jnp.where(pos < l, v, 0.0))

        acc = lax.fori_loop(0, NCHUNK, fused, jnp.zeros((LANES,), jnp.float32),
                            unroll=8)
        outvec = jnp.where(iota == i, _lanesum_splat(acc), outvec)

    out_v[...] = outvec
    pltpu.sync_copy(out_v.at[pl.ds(0, SPW)], out_hbm.at[pl.ds(base_b, SPW)])


def kernel(feats, weights, tags, length):
    return _score(feats.reshape(-1), weights, tags, length)


# speedup vs baseline: 2.0902x; 1.0257x over previous
import functools

import jax
import jax.numpy as jnp
from jax import lax
from jax.experimental import pallas as pl
from jax.experimental.pallas import tpu as pltpu
from jax.experimental.pallas import tpu_sc as plsc

STATE = 128
START = 126
END = 127
B = 256
L = 512
LANES = 16
NC, NS = 2, 16
NW = NC * NS
SPW = B // NW
NGATHER = 8
GLEN = L // NGATHER
CHUNKS_PER_G = GLEN // LANES
NCHUNK = L // LANES
UNROLL = 4

_GATHER_DNUMS = lax.GatherDimensionNumbers(
    offset_dims=(), collapsed_slice_dims=(0,), start_index_map=(0,))


def _permute(v, idx):
    return lax.gather(v, idx[:, None], _GATHER_DNUMS, slice_sizes=(1,),
                      mode=lax.GatherScatterMode.PROMISE_IN_BOUNDS)


def _lanesum_splat(v):
    for sh in (8, 4, 2, 1):
        v = v + _permute(v, jnp.bitwise_xor(lax.iota(jnp.int32, LANES), sh))
    return v


@functools.partial(
    pl.kernel,
    out_type=jax.ShapeDtypeStruct((B,), jnp.float32),
    mesh=plsc.VectorSubcoreMesh(core_axis_name="c", subcore_axis_name="s"),
    compiler_params=pltpu.CompilerParams(needs_layout_passes=False),
    scratch_types=[
        pltpu.VMEM((STATE, STATE), jnp.float32),
        pltpu.VMEM((2 * SPW, L), jnp.int32),
        pltpu.VMEM((2, NGATHER, GLEN), jnp.int32),
        pltpu.VMEM((2, L), jnp.float32),
        pltpu.VMEM((LANES,), jnp.int32),
        pltpu.VMEM((LANES,), jnp.float32),
        pltpu.SemaphoreType.DMA,
        pltpu.SemaphoreType.DMA,
    ],
)
def _score(feats_hbm, w_hbm, tags_hbm, len_hbm, out_hbm,
           w_v, tags_v, idx_v, vals_v, len_v, out_v, sem0, sem1):
    wid = lax.axis_index("s") * NC + lax.axis_index("c")
    base_b = wid * SPW
    iota = lax.iota(jnp.int32, LANES)
    iota_s = iota * STATE
    sems = (sem0, sem1)
    pltpu.sync_copy(w_hbm, w_v)
    pltpu.sync_copy(len_hbm.at[pl.ds(base_b, SPW)], len_v.at[pl.ds(0, SPW)])
    pltpu.sync_copy(tags_hbm.at[pl.ds(base_b, SPW)], tags_v.at[pl.ds(SPW, SPW)])
    lenvec = len_v[...].astype(jnp.float32)

    def build_and_fire(i, p):
        fbase = (base_b + i) * (L * STATE)
        for g in range(NGATHER):
            gbase = fbase + g * (GLEN * STATE)

            def build(c, carry, g=g, gbase=gbase):
                off = c * LANES
                tt = tags_v[i + SPW, pl.ds(g * GLEN + off, LANES)]
                idx_v[p, g, pl.ds(off, LANES)] = (gbase + off * STATE) + (iota_s + tt)
                return carry

            lax.fori_loop(0, CHUNKS_PER_G, build, 0, unroll=4)
        return [
            pltpu.async_copy(feats_hbm.at[idx_v.at[p, g]],
                             vals_v.at[p, pl.ds(g * GLEN, GLEN)], sems[p])
            for g in range(NGATHER)
        ]

    copies = {0: build_and_fire(0, 0)}
    outvec = jnp.zeros((LANES,), jnp.float32)
    for i in range(SPW):
        p = i % 2
        if i + 1 < SPW:
            copies[i + 1] = build_and_fire(i + 1, 1 - p)

        l = _lanesum_splat(jnp.where(iota == i, lenvec, 0.0)).astype(jnp.int32)

        for cp in copies.pop(i):
            cp.wait()

        rowidx = jnp.full((LANES,), i + SPW, jnp.int32)

        def fused(c, acc, p=p, rowidx=rowidx, l=l, i=i):
            pos = c * LANES + iota
            tt = tags_v[i + SPW, pl.ds(c * LANES, LANES)]
            prev = plsc.load_gather(tags_v, [rowidx, jnp.maximum(pos - 1, 0)])
            col = jnp.where(pos == 0, START, prev)
            row = jnp.where(pos == l, END, tt)
            wv = plsc.load_gather(w_v, [row, col])
            v = vals_v[p, pl.ds(c * LANES, LANES)]
            return (acc + jnp.where(pos <= l, wv, 0.0)
                    + jnp.where(pos < l, v, 0.0))

        acc = lax.fori_loop(0, NCHUNK, fused, jnp.zeros((LANES,), jnp.float32),
                            unroll=4)
        outvec = jnp.where(iota == i, _lanesum_splat(acc), outvec)

    out_v[...] = outvec
    pltpu.sync_copy(out_v.at[pl.ds(0, SPW)], out_hbm.at[pl.ds(base_b, SPW)])


def kernel(feats, weights, tags, length):
    return _score(feats.reshape(-1), weights, tags, length)


# speedup vs baseline: 2.2146x; 1.0595x over previous
import functools

import jax
import jax.numpy as jnp
from jax import lax
from jax.experimental import pallas as pl
from jax.experimental.pallas import tpu as pltpu
from jax.experimental.pallas import tpu_sc as plsc

STATE = 128
START = 126
END = 127
B = 256
L = 512
LANES = 16
NC, NS = 2, 16
NW = NC * NS
SPW = B // NW
NGATHER = 4
GLEN = L // NGATHER
CHUNKS_PER_G = GLEN // LANES
NCHUNK = L // LANES
UNROLL = 4

_GATHER_DNUMS = lax.GatherDimensionNumbers(
    offset_dims=(), collapsed_slice_dims=(0,), start_index_map=(0,))


def _permute(v, idx):
    return lax.gather(v, idx[:, None], _GATHER_DNUMS, slice_sizes=(1,),
                      mode=lax.GatherScatterMode.PROMISE_IN_BOUNDS)


def _lanesum_splat(v):
    for sh in (8, 4, 2, 1):
        v = v + _permute(v, jnp.bitwise_xor(lax.iota(jnp.int32, LANES), sh))
    return v


@functools.partial(
    pl.kernel,
    out_type=jax.ShapeDtypeStruct((B,), jnp.float32),
    mesh=plsc.VectorSubcoreMesh(core_axis_name="c", subcore_axis_name="s"),
    compiler_params=pltpu.CompilerParams(needs_layout_passes=False),
    scratch_types=[
        pltpu.VMEM((STATE, STATE), jnp.float32),
        pltpu.VMEM((2 * SPW, L), jnp.int32),
        pltpu.VMEM((SPW, NGATHER, GLEN), jnp.int32),
        pltpu.VMEM((SPW, L), jnp.float32),
        pltpu.VMEM((LANES,), jnp.int32),
        pltpu.VMEM((LANES,), jnp.float32),
    ] + [pltpu.SemaphoreType.DMA] * SPW,
)
def _score(feats_hbm, w_hbm, tags_hbm, len_hbm, out_hbm,
           w_v, tags_v, idx_v, vals_v, len_v, out_v, *sems):
    wid = lax.axis_index("s") * NC + lax.axis_index("c")
    base_b = wid * SPW
    iota = lax.iota(jnp.int32, LANES)
    iota_s = iota * STATE
    pltpu.sync_copy(w_hbm, w_v)
    pltpu.sync_copy(len_hbm.at[pl.ds(base_b, SPW)], len_v.at[pl.ds(0, SPW)])
    pltpu.sync_copy(tags_hbm.at[pl.ds(base_b, SPW)], tags_v.at[pl.ds(SPW, SPW)])
    lenvec = len_v[...].astype(jnp.float32)

    def build_and_fire(i):
        fbase = (base_b + i) * (L * STATE)
        for g in range(NGATHER):
            gbase = fbase + g * (GLEN * STATE)

            def build(c, carry, g=g, gbase=gbase):
                off = c * LANES
                tt = tags_v[i + SPW, pl.ds(g * GLEN + off, LANES)]
                idx_v[i, g, pl.ds(off, LANES)] = (gbase + off * STATE) + (iota_s + tt)
                return carry

            lax.fori_loop(0, CHUNKS_PER_G, build, 0, unroll=4)
        return [
            pltpu.async_copy(feats_hbm.at[idx_v.at[i, g]],
                             vals_v.at[i, pl.ds(g * GLEN, GLEN)], sems[i])
            for g in range(NGATHER)
        ]

    copies = [build_and_fire(i) for i in range(SPW)]
    outvec = jnp.zeros((LANES,), jnp.float32)
    for i in range(SPW):
        l = _lanesum_splat(jnp.where(iota == i, lenvec, 0.0)).astype(jnp.int32)

        for cp in copies[i]:
            cp.wait()

        rowidx = jnp.full((LANES,), i + SPW, jnp.int32)

        def fused(c, acc, rowidx=rowidx, l=l, i=i):
            pos = c * LANES + iota
            tt = tags_v[i + SPW, pl.ds(c * LANES, LANES)]
            prev = plsc.load_gather(tags_v, [rowidx, jnp.maximum(pos - 1, 0)])
            col = jnp.where(pos == 0, START, prev)
            row = jnp.where(pos == l, END, tt)
            wv = plsc.load_gather(w_v, [row, col])
            v = vals_v[i, pl.ds(c * LANES, LANES)]
            return (acc + jnp.where(pos <= l, wv, 0.0)
                    + jnp.where(pos < l, v, 0.0))

        acc = lax.fori_loop(0, NCHUNK, fused, jnp.zeros((LANES,), jnp.float32),
                            unroll=4)
        outvec = jnp.where(iota == i, _lanesum_splat(acc), outvec)

    out_v[...] = outvec
    pltpu.sync_copy(out_v.at[pl.ds(0, SPW)], out_hbm.at[pl.ds(base_b, SPW)])


def kernel(feats, weights, tags, length):
    return _score(feats.reshape(-1), weights, tags, length)


# speedup vs baseline: 2.3109x; 1.0435x over previous
import functools

import jax
import jax.numpy as jnp
from jax import lax
from jax.experimental import pallas as pl
from jax.experimental.pallas import tpu as pltpu
from jax.experimental.pallas import tpu_sc as plsc

STATE = 128
START = 126
END = 127
B = 256
L = 512
LANES = 16
NC, NS = 2, 16
NW = NC * NS
SPW = B // NW
NGATHER = 4
GLEN = L // NGATHER
CHUNKS_PER_G = GLEN // LANES
NCHUNK = L // LANES
UNROLL = 4

_GATHER_DNUMS = lax.GatherDimensionNumbers(
    offset_dims=(), collapsed_slice_dims=(0,), start_index_map=(0,))


def _permute(v, idx):
    return lax.gather(v, idx[:, None], _GATHER_DNUMS, slice_sizes=(1,),
                      mode=lax.GatherScatterMode.PROMISE_IN_BOUNDS)


def _lanesum_splat(v):
    for sh in (8, 4, 2, 1):
        v = v + _permute(v, jnp.bitwise_xor(lax.iota(jnp.int32, LANES), sh))
    return v


@functools.partial(
    pl.kernel,
    out_type=jax.ShapeDtypeStruct((B,), jnp.float32),
    mesh=plsc.VectorSubcoreMesh(core_axis_name="c", subcore_axis_name="s"),
    compiler_params=pltpu.CompilerParams(needs_layout_passes=False),
    scratch_types=[
        pltpu.VMEM((STATE, STATE), jnp.float32),
        pltpu.VMEM((2 * SPW, L), jnp.int32),
        pltpu.VMEM((SPW, NGATHER, GLEN), jnp.int32),
        pltpu.VMEM((SPW, L), jnp.float32),
        pltpu.VMEM((LANES,), jnp.int32),
        pltpu.VMEM((LANES,), jnp.float32),
    ] + [pltpu.SemaphoreType.DMA] * (SPW + 1),
)
def _score(feats_hbm, w_hbm, tags_hbm, len_hbm, out_hbm,
           w_v, tags_v, idx_v, vals_v, len_v, out_v, *sems):
    wid = lax.axis_index("s") * NC + lax.axis_index("c")
    base_b = wid * SPW
    iota = lax.iota(jnp.int32, LANES)
    iota_s = iota * STATE
    pltpu.sync_copy(tags_hbm.at[pl.ds(base_b, SPW)], tags_v.at[pl.ds(SPW, SPW)])
    w_copy = pltpu.async_copy(w_hbm, w_v, sems[SPW])
    pltpu.sync_copy(len_hbm.at[pl.ds(base_b, SPW)], len_v.at[pl.ds(0, SPW)])
    lenvec = len_v[...].astype(jnp.float32)

    def build_and_fire(i):
        fbase = (base_b + i) * (L * STATE)
        for g in range(NGATHER):
            gbase = fbase + g * (GLEN * STATE)

            def build(c, carry, g=g, gbase=gbase):
                off = c * LANES
                tt = tags_v[i + SPW, pl.ds(g * GLEN + off, LANES)]
                idx_v[i, g, pl.ds(off, LANES)] = (gbase + off * STATE) + (iota_s + tt)
                return carry

            lax.fori_loop(0, CHUNKS_PER_G, build, 0, unroll=4)
        return [
            pltpu.async_copy(feats_hbm.at[idx_v.at[i, g]],
                             vals_v.at[i, pl.ds(g * GLEN, GLEN)], sems[i])
            for g in range(NGATHER)
        ]

    copies = [build_and_fire(i) for i in range(SPW)]
    w_copy.wait()
    outvec = jnp.zeros((LANES,), jnp.float32)
    for i in range(SPW):
        l = _lanesum_splat(jnp.where(iota == i, lenvec, 0.0)).astype(jnp.int32)
        rowidx = jnp.full((LANES,), i + SPW, jnp.int32)
        plsc.store_scatter(tags_v, [rowidx, l],
                           jnp.full((LANES,), END, jnp.int32), mask=iota == 0)

        for cp in copies[i]:
            cp.wait()

        def fused(c, acc, rowidx=rowidx, l=l, i=i):
            pos = c * LANES + iota
            tt = tags_v[i + SPW, pl.ds(c * LANES, LANES)]
            prev = plsc.load_gather(tags_v, [rowidx, jnp.maximum(pos - 1, 0)])
            col = jnp.where(pos == 0, START, prev)
            wv = plsc.load_gather(w_v, [tt, col])
            v = vals_v[i, pl.ds(c * LANES, LANES)]
            return (acc + jnp.where(pos <= l, wv, 0.0)
                    + jnp.where(pos < l, v, 0.0))

        acc = lax.fori_loop(0, NCHUNK, fused, jnp.zeros((LANES,), jnp.float32),
                            unroll=4)
        outvec = jnp.where(iota == i, _lanesum_splat(acc), outvec)

    out_v[...] = outvec
    pltpu.sync_copy(out_v.at[pl.ds(0, SPW)], out_hbm.at[pl.ds(base_b, SPW)])


def kernel(feats, weights, tags, length):
    return _score(feats.reshape(-1), weights, tags, length)
